# baseline (speedup 1.0000x reference)
import functools

import jax
import jax.numpy as jnp
from jax import lax
from jax.experimental import pallas as pl
from jax.experimental.pallas import tpu as pltpu
from jax.experimental.pallas import tpu_sc as plsc

N = 10000
E = 320000
D = 128

NS = 16
CHUNK = 128
CPW = 160
QC = CPW // 4
TCH = NS * CPW
E_PAD = TCH * CHUNK
NPAD = 10112
RPS = NPAD // NS


def _sc_aggregate(src2, dst2, xp):
    mesh = plsc.VectorSubcoreMesh(core_axis_name="c", subcore_axis_name="s")

    @functools.partial(
        pl.kernel,
        out_type=jax.ShapeDtypeStruct((NPAD, D), jnp.float32),
        mesh=mesh,
        scratch_types=[
            pltpu.VMEM((QC, CHUNK), jnp.int32),
            pltpu.VMEM((QC, CHUNK), jnp.int32),
            [pltpu.VMEM((CHUNK, D), jnp.float32) for _ in range(2)],
            pltpu.VMEM_SHARED((NPAD, D), jnp.float32),
            [pltpu.SemaphoreType.DMA for _ in range(2)],
            [pltpu.SemaphoreType.DMA for _ in range(2)],
        ],
    )
    def agg_kernel(src_hbm, dst_hbm, x_hbm, out_hbm, src_v, dst_v, bufs,
                   agg_sh, gsems, ssems):
        c = lax.axis_index("c")
        s = lax.axis_index("s")

        def slot(k, b):
            pltpu.make_async_copy(x_hbm.at[src_v.at[k]], bufs[b],
                                  gsems[b]).wait()
            pltpu.async_copy(bufs[b], agg_sh.at[dst_v.at[k]], ssems[b],
                             add=True)

            @pl.when(k + 1 < QC)
            def _():
                @pl.when(k >= 1)
                def _():
                    pltpu.make_async_copy(bufs[1 - b],
                                          agg_sh.at[dst_v.at[k - 1]],
                                          ssems[1 - b]).wait()

                pltpu.async_copy(x_hbm.at[src_v.at[k + 1]], bufs[1 - b],
                                 gsems[1 - b])

        @pl.when(c == 0)
        def _():
            base = s * RPS
            pltpu.sync_copy(x_hbm.at[pl.ds(base, RPS)],
                            agg_sh.at[pl.ds(base, RPS)])
            plsc.subcore_barrier()

            for q in range(4):
                start = s * CPW + q * QC
                pltpu.sync_copy(src_hbm.at[pl.ds(start, QC)], src_v)
                pltpu.sync_copy(dst_hbm.at[pl.ds(start, QC)], dst_v)
                pltpu.async_copy(x_hbm.at[src_v.at[0]], bufs[0], gsems[0])

                @pl.loop(0, QC, step=2)
                def _(j):
                    slot(j, 0)
                    slot(j + 1, 1)

                pltpu.make_async_copy(bufs[0], agg_sh.at[dst_v.at[QC - 2]],
                                      ssems[0]).wait()
                pltpu.make_async_copy(bufs[1], agg_sh.at[dst_v.at[QC - 1]],
                                      ssems[1]).wait()

            plsc.subcore_barrier()
            pltpu.sync_copy(agg_sh.at[pl.ds(base, RPS)],
                            out_hbm.at[pl.ds(base, RPS)])

    return agg_kernel(src2, dst2, xp)


def _mlp_body(p_ref, w1_ref, b1_ref, w2_ref, b2_ref, o_ref):
    h = jnp.maximum(
        jnp.dot(p_ref[...], w1_ref[...], preferred_element_type=jnp.float32)
        + b1_ref[...], 0.0)
    o_ref[...] = (jnp.dot(h, w2_ref[...], preferred_element_type=jnp.float32)
                  + b2_ref[...])


def _mlp(agg, W1, b1, W2, b2):
    BLK = 1000
    grid = (N // BLK,)
    return pl.pallas_call(
        _mlp_body,
        grid=grid,
        in_specs=[
            pl.BlockSpec((BLK, D), lambda i: (i, 0)),
            pl.BlockSpec((D, D), lambda i: (0, 0)),
            pl.BlockSpec((1, D), lambda i: (0, 0)),
            pl.BlockSpec((D, D), lambda i: (0, 0)),
            pl.BlockSpec((1, D), lambda i: (0, 0)),
        ],
        out_specs=pl.BlockSpec((BLK, D), lambda i: (i, 0)),
        out_shape=jax.ShapeDtypeStruct((N, D), jnp.float32),
    )(agg, W1, b1, W2, b2)


@jax.jit
def kernel(x, edge_index, W1, b1, W2, b2):
    src = edge_index[0]
    dst = edge_index[1]
    pad = E_PAD - E
    src_p = jnp.concatenate([src, jnp.zeros((pad,), jnp.int32)])
    dst_p = jnp.concatenate([dst, jnp.full((pad,), N, jnp.int32)])
    src2 = src_p.reshape(TCH, CHUNK)
    dst2 = dst_p.reshape(TCH, CHUNK)
    xp = jnp.concatenate([x, jnp.zeros((NPAD - N, D), jnp.float32)])

    agg = _sc_aggregate(src2, dst2, xp)
    return _mlp(agg, W1, b1.reshape(1, D), W2, b2.reshape(1, D))
